# baseline (speedup 1.0000x reference)
import functools

import jax
import jax.numpy as jnp
from jax import lax
from jax.experimental import pallas as pl
from jax.experimental.pallas import tpu as pltpu
from jax.experimental.pallas import tpu_sc as plsc

B, T, D = 4096, 200, 128
OUT_LEN = 50
L = 16
NC, NS = 2, 16
NW = NC * NS
BPW = B // NW

_mesh = plsc.VectorSubcoreMesh(core_axis_name="c", subcore_axis_name="s")


@functools.partial(
    pl.kernel,
    mesh=_mesh,
    out_type=jax.ShapeDtypeStruct((B, D), jnp.float32),
    scratch_types=[
        pltpu.VMEM((BPW,), jnp.int32),
        pltpu.VMEM((BPW,), jnp.int32),
        pltpu.VMEM((BPW, D), jnp.float32),
        pltpu.SemaphoreType.DMA,
    ],
)
def _gather_last(x_hbm, sl_hbm, out_hbm, sl_v, idx_v, rows_v, gsem):
    wid = lax.axis_index("s") * NC + lax.axis_index("c")
    base = wid * BPW

    pltpu.sync_copy(sl_hbm.at[pl.ds(base, BPW)], sl_v)

    for i in range(BPW // L):
        s = sl_v[pl.ds(i * L, L)]
        t = jnp.where(s == 0, T - 1, s - 1)
        row = (base + i * L) + lax.iota(jnp.int32, L)
        idx_v[pl.ds(i * L, L)] = row * T + t

    pltpu.async_copy(x_hbm.at[idx_v], rows_v, gsem).wait()
    pltpu.sync_copy(rows_v, out_hbm.at[pl.ds(base, BPW)])


def kernel(x, seq_len, out_len):
    del out_len
    g = _gather_last(x.reshape(B * T, D), seq_len.astype(jnp.int32))
    return jnp.broadcast_to(g[:, None, :], (B, OUT_LEN, D))
